# Initial kernel scaffold; baseline (speedup 1.0000x reference)
import jax
import jax.numpy as jnp
from jax.experimental import pallas as pl


def kernel(x, edge_index, W1l, W1r, b1, W2l, W2r, b2, Wo, bo):
    raise NotImplementedError("write your pallas kernel here")


# speedup vs baseline: 4.3303x; 4.3303x over previous
import dataclasses
import functools

import jax
import jax.numpy as jnp
from jax import lax
from jax.experimental import pallas as pl
from jax.experimental.pallas import tpu as pltpu
from jax.experimental.pallas import tpu_sc as plsc

N = 10000
D = 128
H = 128
NC = 2
NS = 16
NW = NC * NS
CHUNK = 128
CPW = 79
PER_W = CPW * CHUNK
E_PAD = NW * PER_W
NP = 10112
RPS = NP // NS
PAD_DST = 10008


def _make_sc_segsum(with_cnt: bool):
    mesh = plsc.VectorSubcoreMesh(core_axis_name="c", subcore_axis_name="s")
    agg_t = jax.ShapeDtypeStruct((NC, NP, H), jnp.float32)
    out_type = [agg_t] if with_cnt else agg_t
    scratch = [
        pltpu.VMEM((CHUNK,), jnp.int32),
        pltpu.VMEM((CHUNK,), jnp.int32),
        pltpu.VMEM((CHUNK, H), jnp.float32),
        pltpu.VMEM_SHARED((NP, H), jnp.float32),
        pltpu.SemaphoreType.DMA,
    ]
    if with_cnt:
        out_type.append(jax.ShapeDtypeStruct((NW * NP,), jnp.float32))
        scratch.append(pltpu.VMEM((NP,), jnp.float32))

    cp = pltpu.CompilerParams()
    if with_cnt and (
            "needs_layout_passes" in pltpu.CompilerParams.__dataclass_fields__):
        cp = dataclasses.replace(cp, needs_layout_passes=False)

    @functools.partial(pl.kernel, out_type=out_type, mesh=mesh,
                       scratch_types=scratch, compiler_params=cp)
    def segsum(feat_hbm, src_hbm, dst_hbm, z_h, *rest):
        if with_cnt:
            (agg_out, cnt_out, src_v, dst_v, rows_v, agg_sh, sem,
             cnt_v) = rest
        else:
            agg_out, src_v, dst_v, rows_v, agg_sh, sem = rest
        cid = lax.axis_index("c")
        sid = lax.axis_index("s")
        wid = cid * NS + sid
        sl = pl.ds(sid * RPS, RPS)

        pltpu.sync_copy(z_h, agg_sh.at[sl])
        if with_cnt:
            zero16 = jnp.zeros((16,), jnp.float32)

            @pl.loop(0, NP // 16)
            def _(i):
                cnt_v[pl.ds(i * 16, 16)] = zero16

        plsc.subcore_barrier()

        base_w = wid * PER_W
        ones16 = jnp.ones((16,), jnp.float32)

        @pl.loop(0, CPW)
        def _(c):
            base = base_w + c * CHUNK
            pltpu.sync_copy(src_hbm.at[pl.ds(base, CHUNK)], src_v)
            pltpu.sync_copy(dst_hbm.at[pl.ds(base, CHUNK)], dst_v)
            pltpu.async_copy(feat_hbm.at[src_v], rows_v, sem).wait()
            pltpu.sync_copy(rows_v, agg_sh.at[dst_v], add=True)
            if with_cnt:
                for j in range(CHUNK // 16):
                    idx16 = dst_v[pl.ds(j * 16, 16)]
                    plsc.addupdate_scatter(cnt_v, [idx16], ones16)

        plsc.subcore_barrier()

        pltpu.sync_copy(agg_sh.at[sl], agg_out.at[cid].at[sl])
        if with_cnt:
            pltpu.sync_copy(cnt_v, cnt_out.at[pl.ds(wid * NP, NP)])

    return segsum


_sc_segsum_cnt = _make_sc_segsum(True)
_sc_segsum = _make_sc_segsum(False)


def _tc_lin(x, w, b):
    def body(x_ref, w_ref, b_ref, o_ref):
        o_ref[...] = jnp.dot(
            x_ref[...], w_ref[...],
            preferred_element_type=jnp.float32,
            precision=lax.Precision.HIGHEST) + b_ref[...]

    return pl.pallas_call(
        body,
        out_shape=jax.ShapeDtypeStruct((x.shape[0], w.shape[1]), jnp.float32),
    )(x, w, b.reshape(1, -1))


def _tc_combine(agg, cnt, wl, other, relu):
    def body(a_ref, c_ref, w_ref, o_ref, out_ref):
        a = a_ref[0, :N, :] + a_ref[1, :N, :]
        c = jnp.sum(c_ref[...], axis=0)[:N]
        mean = a / jnp.clip(c, 1.0)[:, None]
        r = jnp.dot(mean, w_ref[...],
                    preferred_element_type=jnp.float32,
                    precision=lax.Precision.HIGHEST) + o_ref[...]
        out_ref[...] = jnp.maximum(r, 0.0) if relu else r

    return pl.pallas_call(
        body,
        out_shape=jax.ShapeDtypeStruct((N, wl.shape[1]), jnp.float32),
    )(agg, cnt, wl, other)


def kernel(x, edge_index, W1l, W1r, b1, W2l, W2r, b2, Wo, bo):
    src = edge_index[0]
    dst = edge_index[1]
    pad = E_PAD - src.shape[0]
    srcp = jnp.concatenate([src, jnp.zeros((pad,), jnp.int32)])
    dstp = jnp.concatenate([dst, jnp.full((pad,), PAD_DST, jnp.int32)])
    z_h = jnp.zeros((RPS, H), jnp.float32)

    agg1, cntp = _sc_segsum_cnt(x, srcp, dstp, z_h)
    cnt = cntp.reshape(NW, NP)
    xr = _tc_lin(x, W1r, b1)
    h = _tc_combine(agg1, cnt, W1l, xr, relu=True)

    agg2 = _sc_segsum(h, srcp, dstp, z_h)
    hr = _tc_lin(h, W2r, b2)
    h2 = _tc_combine(agg2, cnt, W2l, hr, relu=True)

    return _tc_lin(h2, Wo, bo)
